# baseline (speedup 1.0000x reference)
import functools

import jax
import jax.numpy as jnp
from jax import lax
from jax.experimental import pallas as pl
from jax.experimental.pallas import tpu as pltpu
from jax.experimental.pallas import tpu_sc as plsc

_NC = 2
_NS = 16
_NW = _NC * _NS

_NB = 4096
_T = 200
_D = 64
_NBUF = 4
_BPW = _NB // _NW * _T
_ROWS_PW = _NB // _NW

_mesh = plsc.VectorSubcoreMesh(core_axis_name="c", subcore_axis_name="s")


@functools.partial(
    pl.kernel,
    mesh=_mesh,
    out_type=jax.ShapeDtypeStruct((_NB, _T, _D), jnp.float32),
    scratch_types=[
        pltpu.VMEM((_BPW,), jnp.int32),
        [pltpu.VMEM((_T, _D), jnp.float32) for _ in range(_NBUF)],
        [pltpu.SemaphoreType.DMA for _ in range(_NBUF)],
        [pltpu.SemaphoreType.DMA for _ in range(_NBUF)],
    ],
    compiler_params=pltpu.CompilerParams(use_tc_tiling_on_sc=False),
)
def _gather_kernel(idx_hbm, table_hbm, out_hbm, idx_v, rows, gsem, osem):
    wid = lax.axis_index("s") * _NC + lax.axis_index("c")
    base = wid * _BPW
    b0 = wid * _ROWS_PW

    pltpu.sync_copy(idx_hbm.at[pl.ds(base, _BPW)], idx_v)

    def _gather(i, q):
        pltpu.async_copy(
            table_hbm.at[idx_v.at[pl.ds(i * _T, _T)]], rows[q], gsem[q])

    def _gather_wait(i, q):
        pltpu.make_async_copy(
            table_hbm.at[idx_v.at[pl.ds(i * _T, _T)]], rows[q], gsem[q]).wait()

    def _store(i, q):
        pltpu.async_copy(rows[q], out_hbm.at[b0 + i], osem[q])

    def _store_wait(i, q):
        pltpu.make_async_copy(rows[q], out_hbm.at[b0 + i], osem[q]).wait()

    for q in range(_NBUF):
        _gather(q, q)

    def outer(k, carry):
        i0 = k * _NBUF
        for q in range(_NBUF):
            i = i0 + q
            _gather_wait(i, q)
            _store(i, q)
            j = i + _NBUF

            @pl.when(j < _ROWS_PW)
            def _():
                _store_wait(i, q)
                _gather(j, q)
        return carry

    lax.fori_loop(0, _ROWS_PW // _NBUF, outer, 0)

    for q in range(_NBUF):
        _store_wait(_ROWS_PW - _NBUF + q, q)


def kernel(tokens, embedding_table):
    idx = tokens.reshape(-1).astype(jnp.int32)
    return _gather_kernel(idx, embedding_table)


# speedup vs baseline: 1.4535x; 1.4535x over previous
import functools

import jax
import jax.numpy as jnp
from jax import lax
from jax.experimental import pallas as pl
from jax.experimental.pallas import tpu as pltpu
from jax.experimental.pallas import tpu_sc as plsc

_NC = 2
_NS = 16
_NW = _NC * _NS
_NB = 4096
_T = 200
_D = 64
_DP = 128
_V = 8192
_B = _NB * _T
_BPW = _B // _NW
_CH = 256
_NCHUNK = _BPW // _CH

_mesh = plsc.VectorSubcoreMesh(core_axis_name="c", subcore_axis_name="s")


@functools.partial(
    pl.kernel,
    mesh=_mesh,
    out_type=jax.ShapeDtypeStruct((_B, _DP), jnp.float32),
    scratch_types=[
        pltpu.VMEM((_BPW,), jnp.int32),
        pltpu.VMEM((_CH, _DP), jnp.float32),
        pltpu.VMEM_SHARED((_V, _DP), jnp.float32),
        pltpu.SemaphoreType.DMA,
    ],
)
def _gather_kernel(idx_hbm, table_hbm, out_hbm, idx_v, rows, table_s, sem):
    wid = lax.axis_index("s") * _NC + lax.axis_index("c")
    base = wid * _BPW

    sid = lax.axis_index("s")
    stripe = _V // _NS
    pltpu.sync_copy(table_hbm.at[pl.ds(sid * stripe, stripe)],
                    table_s.at[pl.ds(sid * stripe, stripe)])
    pltpu.sync_copy(idx_hbm.at[pl.ds(base, _BPW)], idx_v)
    plsc.subcore_barrier()

    def body(i, carry):
        pltpu.async_copy(
            table_s.at[idx_v.at[pl.ds(i * _CH, _CH)]], rows, sem)
        pltpu.make_async_copy(
            table_s.at[idx_v.at[pl.ds(i * _CH, _CH)]], rows, sem).wait()
        pltpu.sync_copy(rows, out_hbm.at[pl.ds(base + i * _CH, _CH)])
        return carry

    lax.fori_loop(0, _NCHUNK, body, 0)


def kernel(tokens, embedding_table):
    idx = tokens.reshape(-1).astype(jnp.int32)
    table_p = jnp.pad(embedding_table, ((0, 0), (0, _DP - _D)))
    out = _gather_kernel(idx, table_p)
    return out[:, :_D].reshape(tokens.shape + (_D,))


# speedup vs baseline: 1.8277x; 1.2574x over previous
import functools

import jax
import jax.numpy as jnp
from jax import lax
from jax.experimental import pallas as pl
from jax.experimental.pallas import tpu as pltpu
from jax.experimental.pallas import tpu_sc as plsc

_NC = 2
_NS = 16
_NW = _NC * _NS
_NB = 4096
_T = 200
_D = 64
_DP = 128
_V = 8192
_B = _NB * _T
_BPW = _B // _NW
_CH = 128
_NCHUNK = _BPW // _CH

_mesh = plsc.VectorSubcoreMesh(core_axis_name="c", subcore_axis_name="s")


@functools.partial(
    pl.kernel,
    mesh=_mesh,
    out_type=jax.ShapeDtypeStruct((_B, _DP), jnp.float32),
    scratch_types=[
        pltpu.VMEM((_BPW,), jnp.int32),
        [pltpu.VMEM((_CH, _DP), jnp.float32) for _ in range(2)],
        pltpu.VMEM_SHARED((_V, _DP), jnp.float32),
        [pltpu.SemaphoreType.DMA for _ in range(2)],
        [pltpu.SemaphoreType.DMA for _ in range(2)],
    ],
)
def _gather_kernel(idx_hbm, table_hbm, out_hbm, idx_v, rows, table_s,
                   gsem, osem):
    wid = lax.axis_index("s") * _NC + lax.axis_index("c")
    base = wid * _BPW

    sid = lax.axis_index("s")
    stripe = _V // _NS
    pltpu.sync_copy(table_hbm.at[pl.ds(sid * stripe, stripe)],
                    table_s.at[pl.ds(sid * stripe, stripe)])
    pltpu.sync_copy(idx_hbm.at[pl.ds(base, _BPW)], idx_v)
    plsc.subcore_barrier()

    def _gather(i, q):
        pltpu.async_copy(
            table_s.at[idx_v.at[pl.ds(i * _CH, _CH)]], rows[q], gsem[q])

    def _gather_wait(i, q):
        pltpu.make_async_copy(
            table_s.at[idx_v.at[pl.ds(i * _CH, _CH)]], rows[q],
            gsem[q]).wait()

    def _store(i, q):
        pltpu.async_copy(rows[q], out_hbm.at[pl.ds(base + i * _CH, _CH)],
                         osem[q])

    def _store_wait(i, q):
        pltpu.make_async_copy(
            rows[q], out_hbm.at[pl.ds(base + i * _CH, _CH)], osem[q]).wait()

    for q in range(2):
        _gather(q, q)

    def outer(k, carry):
        i0 = k * 2
        for q in range(2):
            i = i0 + q
            _gather_wait(i, q)
            _store(i, q)
            j = i + 2

            @pl.when(j < _NCHUNK)
            def _():
                _store_wait(i, q)
                _gather(j, q)
        return carry

    lax.fori_loop(0, _NCHUNK // 2, outer, 0)

    for q in range(2):
        _store_wait(_NCHUNK - 2 + q, q)


def kernel(tokens, embedding_table):
    idx = tokens.reshape(-1).astype(jnp.int32)
    table_p = jnp.pad(embedding_table, ((0, 0), (0, _DP - _D)))
    out = _gather_kernel(idx, table_p)
    return out[:, :_D].reshape(tokens.shape + (_D,))
